# baseline (speedup 1.0000x reference)
import jax
import jax.numpy as jnp
from jax.experimental import pallas as pl
from jax.experimental.pallas import tpu as pltpu

CHANNELS = 768
TOPK = 384
TEMP = 1.0
NB = 16
XDIM = 56 * 56
ROWS = NB * CHANNELS
R_BLK = 256
N_RBLK = ROWS // R_BLK
M_PERIOD = CHANNELS // R_BLK if R_BLK <= CHANNELS else 1


def _mask_kernel(logits_ref, m_ref):
    lg = logits_ref[0, :]
    col = lg[None, :]
    row = lg[:, None]
    i_idx = jax.lax.broadcasted_iota(jnp.int32, (CHANNELS, CHANNELS), 0)
    j_idx = jax.lax.broadcasted_iota(jnp.int32, (CHANNELS, CHANNELS), 1)
    beats = (col > row) | ((col == row) & (j_idx < i_idx))
    rank = jnp.sum(beats.astype(jnp.int32), axis=1)
    hard = (rank < TOPK).astype(jnp.float32)
    soft = jax.nn.sigmoid(lg / TEMP)
    m = (hard - soft) + soft
    m_ref[0, :] = m


def _gate_kernel(z_ref, m_ref, out_ref):
    out_ref[...] = z_ref[...] * m_ref[0]


def kernel(z, logits):
    z2 = z.reshape(ROWS, XDIM)
    m_out = pl.pallas_call(
        _mask_kernel,
        out_shape=jax.ShapeDtypeStruct((1, CHANNELS), jnp.float32),
    )(logits.reshape(1, CHANNELS))
    m3 = m_out.reshape(M_PERIOD, R_BLK, 1)

    out = pl.pallas_call(
        _gate_kernel,
        grid=(N_RBLK,),
        in_specs=[
            pl.BlockSpec((R_BLK, XDIM), lambda i: (i, 0)),
            pl.BlockSpec((1, R_BLK, 1), lambda i: (i % M_PERIOD, 0, 0)),
        ],
        out_specs=pl.BlockSpec((R_BLK, XDIM), lambda i: (i, 0)),
        out_shape=jax.ShapeDtypeStruct((ROWS, XDIM), jnp.float32),
    )(z2, m3)
    return out.reshape(z.shape)
